# baseline (speedup 1.0000x reference)
import functools

import jax
import jax.numpy as jnp
from jax import lax
from jax.experimental import pallas as pl
from jax.experimental.pallas import tpu as pltpu
from jax.experimental.pallas import tpu_sc as plsc

_NUM = {"user": 10000, "track": 50000, "artist": 10000, "tag": 2000}
_DST_PAD = {"user": 10112, "track": 50176, "artist": 10112, "tag": 2176}
_EDGES = [
    ("ut", "user", "track"),
    ("tu", "track", "user"),
    ("ta", "track", "artist"),
    ("at", "artist", "track"),
    ("tg", "track", "tag"),
    ("gt", "tag", "track"),
]
_ECOUNT = {"ut": 160000, "tu": 160000, "ta": 50000, "at": 50000,
           "tg": 100000, "gt": 100000}
_EPAD = {"ut": 163840, "tu": 163840, "ta": 57344, "at": 57344,
         "tg": 106496, "gt": 106496}
_NB = {en: _EPAD[en] // 64 for en in _EPAD}

_MESH = plsc.VectorSubcoreMesh(core_axis_name="c", subcore_axis_name="s")
_F32 = jnp.float32


def _zero_fill(buf, rows, cols):
    z = jnp.zeros((16,), _F32)
    def zb(r, _):
        for cc in range(cols // 16):
            buf[r, pl.ds(cc * 16, 16)] = z
        return 0
    lax.fori_loop(0, rows, zb, 0)


def _zero_acc(buf, acc, base, chunks, sem):
    for ofs, ln in chunks:
        pltpu.async_copy(buf.at[pl.ds(0, ln)], acc.at[pl.ds(base + ofs, ln)],
                         sem)
    for ofs, ln in chunks:
        pltpu.make_async_copy(buf.at[pl.ds(0, ln)],
                              acc.at[pl.ds(base + ofs, ln)], sem).wait()


def _pipe(tbl, acc, idxs, idxd, nbt, bufs, sgs, sss):
    nr = len(bufs)
    def gfire(g, i):
        pltpu.async_copy(tbl.at[idxs.at[g]], bufs[i], sgs[i])
    def gwait(i):
        pltpu.make_async_copy(tbl.at[idxs.at[0]], bufs[i], sgs[i]).wait()
    def sfire(g, i):
        pltpu.async_copy(bufs[i], acc.at[idxd.at[g]], sss[i], add=True)
    def swait(i):
        pltpu.make_async_copy(bufs[i], acc.at[idxd.at[0]], sss[i]).wait()

    ngrp = nbt // nr
    for i in range(nr):
        gfire(i, i)
    def body(k, _):
        for i in range(nr):
            gwait(i)
            sfire(k * nr + i, i)
        @pl.when(k < ngrp - 1)
        def _():
            for i in range(nr):
                swait(i)
                gfire((k + 1) * nr + i, i)
        return 0
    lax.fori_loop(0, ngrp, body, 0)
    for i in range(nr):
        swait(i)


_CNT_PHASES = [(en, _DST_PAD[dt]) for en, _, dt in _EDGES]


@functools.partial(
    pl.kernel,
    out_type=[jax.ShapeDtypeStruct((2, 50176, 16), _F32) for _ in _CNT_PHASES],
    mesh=_MESH,
    compiler_params=pltpu.CompilerParams(use_tc_tiling_on_sc=False),
    scratch_types=[
        pltpu.VMEM_SHARED((50176, 16), _F32),
        pltpu.VMEM((196, 16), _F32),
        pltpu.VMEM((64, 16), _F32),
        pltpu.VMEM((80, 64), jnp.int32),
        pltpu.SemaphoreType.DMA,
        pltpu.SemaphoreType.DMA,
    ],
)
def _sc_counts(z_h, o_h, d_ut, d_tu, d_ta, d_at, d_tg, d_gt,
               o_ut, o_tu, o_ta, o_at, o_tg, o_gt,
               acc, zbuf, ones, idxd, sz, ss):
    c = lax.axis_index("c")
    s = lax.axis_index("s")
    w = s * 2 + c
    pltpu.sync_copy(z_h, zbuf)
    pltpu.sync_copy(o_h, ones)
    drefs = {"ut": d_ut, "tu": d_tu, "ta": d_ta, "at": d_at,
             "tg": d_tg, "gt": d_gt}
    orefs = {"ut": o_ut, "tu": o_tu, "ta": o_ta, "at": o_at,
             "tg": o_tg, "gt": o_gt}
    def zf(i, _):
        pltpu.async_copy(zbuf, acc.at[pl.ds((s * 16 + i) * 196, 196)], sz)
        return 0
    lax.fori_loop(0, 16, zf, 0)
    def zd(i, _):
        pltpu.make_async_copy(zbuf, acc.at[pl.ds(s * 3136, 196)], sz).wait()
        return 0
    lax.fori_loop(0, 16, zd, 0)
    plsc.subcore_barrier()
    for en, dst_pad in _CNT_PHASES:
        nbt = _NB[en] // 32
        pltpu.sync_copy(drefs[en].at[w], idxd.at[pl.ds(0, nbt)])
        def sf(g, _, ones=ones):
            pltpu.async_copy(ones, acc.at[idxd.at[g]], ss, add=True)
            return 0
        lax.fori_loop(0, nbt, sf, 0)
        def sd(g, _, ones=ones):
            pltpu.make_async_copy(ones, acc.at[idxd.at[0]], ss).wait()
            return 0
        lax.fori_loop(0, nbt, sd, 0)
        plsc.subcore_barrier()
        pltpu.sync_copy(acc.at[pl.ds(s * 3136, 3136)],
                        orefs[en].at[c, pl.ds(s * 3136, 3136)])
        plsc.subcore_barrier()


_UAT_PHASES = [("tu", "user"), ("ta", "artist"), ("tg", "tag")]


@functools.partial(
    pl.kernel,
    out_type=[jax.ShapeDtypeStruct((2, _DST_PAD[dt], 128), _F32)
              for _, dt in _UAT_PHASES],
    mesh=_MESH,
    compiler_params=pltpu.CompilerParams(use_tc_tiling_on_sc=False),
    scratch_types=[
        pltpu.VMEM_SHARED((10112, 128), _F32),
        pltpu.VMEM((64, 128), _F32),
        pltpu.VMEM((64, 128), _F32),
        pltpu.VMEM((64, 128), _F32),
        pltpu.VMEM((64, 128), _F32),
        pltpu.VMEM((80, 64), jnp.int32),
        pltpu.VMEM((80, 64), jnp.int32),
        pltpu.SemaphoreType.DMA,
        pltpu.SemaphoreType.DMA,
        pltpu.SemaphoreType.DMA,
        pltpu.SemaphoreType.DMA,
        pltpu.SemaphoreType.DMA,
        pltpu.SemaphoreType.DMA,
        pltpu.SemaphoreType.DMA,
        pltpu.SemaphoreType.DMA,
        pltpu.SemaphoreType.DMA,
    ],
)
def _sc_uat(tbl, s_tu, d_tu, s_ta, d_ta, s_tg, d_tg,
            o_tu, o_ta, o_tg,
            acc, b0, b1, b2, b3, idxs, idxd,
            sz, sg0, sg1, sg2, sg3, ss0, ss1, ss2, ss3):
    c = lax.axis_index("c")
    s = lax.axis_index("s")
    w = s * 2 + c
    bufs = [b0, b1, b2, b3]
    sgs = [sg0, sg1, sg2, sg3]
    sss = [ss0, ss1, ss2, ss3]
    srefs = {"tu": (s_tu, d_tu, o_tu), "ta": (s_ta, d_ta, o_ta),
             "tg": (s_tg, d_tg, o_tg)}
    zchunks = [(0, 64), (64, 64), (128, 64), (192, 64), (256, 64), (320, 64),
               (384, 64), (448, 64), (512, 64), (576, 56)]
    _zero_fill(b0, 64, 128)
    _zero_acc(b0, acc, s * 632, zchunks, sz)
    plsc.subcore_barrier()
    for en, dt in _UAT_PHASES:
        sref, dref, oref = srefs[en]
        nbt = _NB[en] // 32
        pltpu.sync_copy(sref.at[w], idxs.at[pl.ds(0, nbt)])
        pltpu.sync_copy(dref.at[w], idxd.at[pl.ds(0, nbt)])
        _pipe(tbl, acc, idxs, idxd, nbt, bufs, sgs, sss)
        plsc.subcore_barrier()
        rpt = _DST_PAD[dt] // 16
        pltpu.sync_copy(acc.at[pl.ds(s * rpt, rpt)],
                        oref.at[c, pl.ds(s * rpt, rpt)])
        plsc.subcore_barrier()


_TRK_PHASES = ["ut", "at", "gt"]


@functools.partial(
    pl.kernel,
    out_type=[jax.ShapeDtypeStruct((4, 50176, 32), _F32) for _ in _TRK_PHASES],
    mesh=_MESH,
    compiler_params=pltpu.CompilerParams(use_tc_tiling_on_sc=False),
    scratch_types=[
        pltpu.VMEM_SHARED((50176, 32), _F32),
        pltpu.VMEM((64, 32), _F32),
        pltpu.VMEM((64, 32), _F32),
        pltpu.VMEM((64, 32), _F32),
        pltpu.VMEM((64, 32), _F32),
        pltpu.VMEM((160, 64), jnp.int32),
        pltpu.VMEM((160, 64), jnp.int32),
        pltpu.SemaphoreType.DMA,
        pltpu.SemaphoreType.DMA,
        pltpu.SemaphoreType.DMA,
        pltpu.SemaphoreType.DMA,
        pltpu.SemaphoreType.DMA,
        pltpu.SemaphoreType.DMA,
        pltpu.SemaphoreType.DMA,
        pltpu.SemaphoreType.DMA,
        pltpu.SemaphoreType.DMA,
    ],
)
def _sc_track(tbl_u, tbl_a, tbl_g,
              s_ut, d_ut, s_at, d_at, s_gt, d_gt,
              o_ut, o_at, o_gt,
              acc, b0, b1, b2, b3, idxs, idxd,
              sz, sg0, sg1, sg2, sg3, ss0, ss1, ss2, ss3):
    c = lax.axis_index("c")
    s = lax.axis_index("s")
    bufs = [b0, b1, b2, b3]
    sgs = [sg0, sg1, sg2, sg3]
    sss = [ss0, ss1, ss2, ss3]
    refs = {"ut": (tbl_u, s_ut, d_ut, o_ut), "at": (tbl_a, s_at, d_at, o_at),
            "gt": (tbl_g, s_gt, d_gt, o_gt)}
    zchunks = [(i * 64, 64) for i in range(49)]
    _zero_fill(b0, 64, 32)
    _zero_acc(b0, acc, s * 3136, zchunks, sz)
    plsc.subcore_barrier()
    for en in _TRK_PHASES:
        tbl, sref, dref, oref = refs[en]
        nbt = _NB[en] // 16
        pltpu.sync_copy(dref.at[s], idxd.at[pl.ds(0, nbt)])
        for j in range(2):
            q = 2 * c + j
            pltpu.sync_copy(sref.at[q, s], idxs.at[pl.ds(0, nbt)])
            _pipe(tbl, acc, idxs, idxd, nbt, bufs, sgs, sss)
            plsc.subcore_barrier()
            pltpu.sync_copy(acc.at[pl.ds(s * 3136, 3136)],
                            oref.at[q, pl.ds(s * 3136, 3136)])
            plsc.subcore_barrier()


_BLK = 1024


def _simple_body(relu, has_pred, *refs):
    if has_pred:
        s0, s1, p0, p1, cinv, x, wl, wr, b, out = refs
        m = (s0[0] + s1[0] - p0[0] - p1[0]) * cinv[...]
    else:
        s0, s1, cinv, x, wl, wr, b, out = refs
        m = (s0[0] + s1[0]) * cinv[...]
    acc = (jnp.dot(m, wl[...], preferred_element_type=_F32)
           + jnp.dot(x[...], wr[...], preferred_element_type=_F32) + b[...])
    if relu:
        acc = jnp.maximum(acc, 0.0)
    out[...] = acc


def _combine_simple(s, pred, cinv, x, wl, wr, b, relu):
    n, d = x.shape
    grid = (pl.cdiv(n, _BLK),)
    row = pl.BlockSpec((_BLK, d), lambda i: (i, 0))
    sspec = [pl.BlockSpec((1, _BLK, d), lambda i: (0, i, 0)),
             pl.BlockSpec((1, _BLK, d), lambda i: (1, i, 0))]
    args = [s, s] + ([pred, pred] if pred is not None else [])
    return pl.pallas_call(
        functools.partial(_simple_body, relu, pred is not None),
        grid=grid,
        in_specs=(sspec * (2 if pred is not None else 1)
                  + [pl.BlockSpec((_BLK, 1), lambda i: (i, 0)),
                     row,
                     pl.BlockSpec((d, d), lambda i: (0, 0)),
                     pl.BlockSpec((d, d), lambda i: (0, 0)),
                     pl.BlockSpec((1, d), lambda i: (0, 0))]),
        out_specs=row,
        out_shape=jax.ShapeDtypeStruct((n, d), _F32),
    )(*args, cinv, x, wl, wr, b)


_TRK_PRED = {
    "ut": [None, ("ut", 0), None, ("ut", 2)],
    "at": [("ut", 1), ("at", 0), ("ut", 3), ("at", 2)],
    "gt": [("at", 1), ("gt", 0), ("at", 3), ("gt", 2)],
}


def _track_body(relu, *refs):
    pos = refs[0:12]
    pred = refs[12:24]
    c_ut, c_at, c_gt, x, wl_ut, wl_at, wl_gt, wr, b, out = refs[24:]
    acc = (jnp.dot(x[...], wr[...], preferred_element_type=_F32) + b[...])
    for e, (en, cinv, wl) in enumerate((("ut", c_ut, wl_ut),
                                        ("at", c_at, wl_at),
                                        ("gt", c_gt, wl_gt))):
        qs = []
        for qi in range(4):
            v = pos[e * 4 + qi][0]
            if _TRK_PRED[en][qi] is not None:
                v = v - pred[e * 4 + qi][0]
            qs.append(v)
        m = jnp.concatenate(qs, axis=1) * cinv[...]
        acc = acc + jnp.dot(m, wl[...], preferred_element_type=_F32)
    acc = acc * (1.0 / 3.0)
    if relu:
        acc = jnp.maximum(acc, 0.0)
    out[...] = acc


def _combine_track(s_ut, s_at, s_gt, c_ut, c_at, c_gt, x,
                   wl_ut, wl_at, wl_gt, wr, b, relu):
    n, d = x.shape
    grid = (pl.cdiv(n, _BLK),)
    row = pl.BlockSpec((_BLK, d), lambda i: (i, 0))
    arrs = {"ut": s_ut, "at": s_at, "gt": s_gt}
    def qspec(q):
        return pl.BlockSpec((1, _BLK, 32), lambda i, q=q: (q, i, 0))
    pos_specs, pos_args, pred_specs, pred_args = [], [], [], []
    for en in ("ut", "at", "gt"):
        for qi in range(4):
            pos_specs.append(qspec(qi))
            pos_args.append(arrs[en])
            p = _TRK_PRED[en][qi]
            if p is None:
                pred_specs.append(qspec(0))
                pred_args.append(arrs[en])
            else:
                pred_specs.append(qspec(p[1]))
                pred_args.append(arrs[p[0]])
    cspec = pl.BlockSpec((_BLK, 1), lambda i: (i, 0))
    wspec = pl.BlockSpec((d, d), lambda i: (0, 0))
    return pl.pallas_call(
        functools.partial(_track_body, relu),
        grid=grid,
        in_specs=(pos_specs + pred_specs
                  + [cspec, cspec, cspec, row, wspec, wspec, wspec, wspec,
                     pl.BlockSpec((1, d), lambda i: (0, 0))]),
        out_specs=row,
        out_shape=jax.ShapeDtypeStruct((n, d), _F32),
    )(*pos_args, *pred_args,
      c_ut, c_at, c_gt, x, wl_ut, wl_at, wl_gt, wr, b)


def _pad2d(a, epad, lo, hi):
    n = epad - a.shape[0]
    pad = lo + jnp.arange(n, dtype=jnp.int32) % (hi - lo)
    return jnp.concatenate([a, pad]).reshape(-1, 64)


def _colsplit(x):
    v = x.shape[0]
    return x.reshape(v, 4, 32).transpose(1, 0, 2).reshape(4 * v, 32)


def kernel(params, user_node_id, track_node_id, artist_node_id, tag_node_id,
           ei_ut, ei_tu, ei_ta, ei_at, ei_tg, ei_gt):
    x = {nt: params["emb_" + nt] for nt in ("user", "track", "artist", "tag")}
    ei = {"ut": ei_ut, "tu": ei_tu, "ta": ei_ta, "at": ei_at,
          "tg": ei_tg, "gt": ei_gt}

    s32, d32, d16, s16q = {}, {}, {}, {}
    for en, srct, dstt in _EDGES:
        sp = _pad2d(ei[en][0], _EPAD[en], 0, _NUM[srct])
        dp = _pad2d(ei[en][1], _EPAD[en], _NUM[dstt], _DST_PAD[dstt])
        s32[en] = sp.reshape(32, -1, 64)
        d32[en] = dp.reshape(32, -1, 64)
        if dstt == "track":
            d16[en] = dp.reshape(16, -1, 64)
            v = _NUM[srct]
            s16q[en] = (sp.reshape(16, -1, 64)[None]
                        + (jnp.arange(4, dtype=jnp.int32) * v)[:, None, None,
                                                               None])

    z16 = jnp.zeros((196, 16), _F32)
    o16 = jnp.ones((64, 16), _F32)

    cnts = _sc_counts(z16, o16, d32["ut"], d32["tu"], d32["ta"], d32["at"],
                      d32["tg"], d32["gt"])
    cinv = {}
    prev = None
    for (en, _, dstt), carr in zip(_EDGES, cnts):
        cur = carr[:, :, 0]
        dcnt = cur if prev is None else cur - prev
        prev = cur
        cnt = (dcnt[0] + dcnt[1])[:_NUM[dstt]]
        cinv[en] = (1.0 / jnp.maximum(cnt, 1.0)).reshape(-1, 1)

    for l in range(2):
        relu = l == 0
        s_ut, s_at, s_gt = _sc_track(
            _colsplit(x["user"]), _colsplit(x["artist"]),
            _colsplit(x["tag"]),
            s16q["ut"], d16["ut"], s16q["at"], d16["at"], s16q["gt"],
            d16["gt"])
        s_tu, s_ta, s_tg = _sc_uat(
            x["track"], s32["tu"], d32["tu"], s32["ta"], d32["ta"],
            s32["tg"], d32["tg"])
        new_x = {}
        new_x["user"] = _combine_simple(
            s_tu, None, cinv["tu"], x["user"], params["l%d_tu_Wl" % l],
            params["l%d_tu_Wr" % l], params["l%d_tu_bl" % l].reshape(1, -1),
            relu)
        new_x["artist"] = _combine_simple(
            s_ta, s_tu, cinv["ta"], x["artist"], params["l%d_ta_Wl" % l],
            params["l%d_ta_Wr" % l], params["l%d_ta_bl" % l].reshape(1, -1),
            relu)
        new_x["tag"] = _combine_simple(
            s_tg, s_ta, cinv["tg"], x["tag"], params["l%d_tg_Wl" % l],
            params["l%d_tg_Wr" % l], params["l%d_tg_bl" % l].reshape(1, -1),
            relu)
        wr_sum = (params["l%d_ut_Wr" % l] + params["l%d_at_Wr" % l]
                  + params["l%d_gt_Wr" % l])
        b_sum = (params["l%d_ut_bl" % l] + params["l%d_at_bl" % l]
                 + params["l%d_gt_bl" % l]).reshape(1, -1)
        new_x["track"] = _combine_track(
            s_ut, s_at, s_gt, cinv["ut"], cinv["at"], cinv["gt"], x["track"],
            params["l%d_ut_Wl" % l], params["l%d_at_Wl" % l],
            params["l%d_gt_Wl" % l], wr_sum, b_sum, relu)
        x = new_x
    return (x["user"], x["track"], x["artist"], x["tag"])


# speedup vs baseline: 1.0010x; 1.0010x over previous
import functools

import jax
import jax.numpy as jnp
from jax import lax
from jax.experimental import pallas as pl
from jax.experimental.pallas import tpu as pltpu
from jax.experimental.pallas import tpu_sc as plsc

_NUM = {"user": 10000, "track": 50000, "artist": 10000, "tag": 2000}
_DST_PAD = {"user": 10112, "track": 50176, "artist": 10112, "tag": 2176}
_EDGES = [
    ("ut", "user", "track"),
    ("tu", "track", "user"),
    ("ta", "track", "artist"),
    ("at", "artist", "track"),
    ("tg", "track", "tag"),
    ("gt", "tag", "track"),
]
_EPAD = {"ut": 163840, "tu": 163840, "ta": 57344, "at": 57344,
         "tg": 106496, "gt": 106496}
_NB = {en: _EPAD[en] // 64 for en in _EPAD}

_MESH = plsc.VectorSubcoreMesh(core_axis_name="c", subcore_axis_name="s")
_F32 = jnp.float32


def _zero_fill(buf, rows, cols):
    z = jnp.zeros((16,), _F32)
    def zb(r, _):
        for cc in range(cols // 16):
            buf[r, pl.ds(cc * 16, 16)] = z
        return 0
    lax.fori_loop(0, rows, zb, 0)


def _zero_acc(buf, acc, base, chunks, sem):
    for ofs, ln in chunks:
        pltpu.async_copy(buf.at[pl.ds(0, ln)], acc.at[pl.ds(base + ofs, ln)],
                         sem)
    for ofs, ln in chunks:
        pltpu.make_async_copy(buf.at[pl.ds(0, ln)],
                              acc.at[pl.ds(base + ofs, ln)], sem).wait()


def _pipe(tbl, acc, idxs, idxd, nbt, bufs, sgs, sss):
    nr = len(bufs)
    def gfire(g, i):
        pltpu.async_copy(tbl.at[idxs.at[g]], bufs[i], sgs[i])
    def gwait(i):
        pltpu.make_async_copy(tbl.at[idxs.at[0]], bufs[i], sgs[i]).wait()
    def sfire(g, i):
        pltpu.async_copy(bufs[i], acc.at[idxd.at[g]], sss[i], add=True)
    def swait(i):
        pltpu.make_async_copy(bufs[i], acc.at[idxd.at[0]], sss[i]).wait()

    ngrp = nbt // nr
    for i in range(nr):
        gfire(i, i)
    def body(k, _):
        for i in range(nr):
            gwait(i)
            sfire(k * nr + i, i)
        @pl.when(k < ngrp - 1)
        def _():
            for i in range(nr):
                swait(i)
                gfire((k + 1) * nr + i, i)
        return 0
    lax.fori_loop(0, ngrp, body, 0)
    for i in range(nr):
        swait(i)


_CNT_PHASES = [(en, _DST_PAD[dt]) for en, _, dt in _EDGES]


@functools.partial(
    pl.kernel,
    out_type=[jax.ShapeDtypeStruct((2, 50176, 16), _F32) for _ in _CNT_PHASES],
    mesh=_MESH,
    compiler_params=pltpu.CompilerParams(use_tc_tiling_on_sc=False),
    scratch_types=[
        pltpu.VMEM_SHARED((50176, 16), _F32),
        pltpu.VMEM((196, 16), _F32),
        pltpu.VMEM((64, 16), _F32),
        pltpu.VMEM((80, 64), jnp.int32),
        pltpu.SemaphoreType.DMA,
        pltpu.SemaphoreType.DMA,
    ],
)
def _sc_counts(z_h, o_h, d_ut, d_tu, d_ta, d_at, d_tg, d_gt,
               o_ut, o_tu, o_ta, o_at, o_tg, o_gt,
               acc, zbuf, ones, idxd, sz, ss):
    c = lax.axis_index("c")
    s = lax.axis_index("s")
    w = s * 2 + c
    pltpu.sync_copy(z_h, zbuf)
    pltpu.sync_copy(o_h, ones)
    drefs = {"ut": d_ut, "tu": d_tu, "ta": d_ta, "at": d_at,
             "tg": d_tg, "gt": d_gt}
    orefs = {"ut": o_ut, "tu": o_tu, "ta": o_ta, "at": o_at,
             "tg": o_tg, "gt": o_gt}
    def zf(i, _):
        pltpu.async_copy(zbuf, acc.at[pl.ds((s * 16 + i) * 196, 196)], sz)
        return 0
    lax.fori_loop(0, 16, zf, 0)
    def zd(i, _):
        pltpu.make_async_copy(zbuf, acc.at[pl.ds(s * 3136, 196)], sz).wait()
        return 0
    lax.fori_loop(0, 16, zd, 0)
    plsc.subcore_barrier()
    for en, dst_pad in _CNT_PHASES:
        nbt = _NB[en] // 32
        pltpu.sync_copy(drefs[en].at[w], idxd.at[pl.ds(0, nbt)])
        def sf(g, _, ones=ones):
            pltpu.async_copy(ones, acc.at[idxd.at[g]], ss, add=True)
            return 0
        lax.fori_loop(0, nbt, sf, 0)
        def sd(g, _, ones=ones):
            pltpu.make_async_copy(ones, acc.at[idxd.at[0]], ss).wait()
            return 0
        lax.fori_loop(0, nbt, sd, 0)
        plsc.subcore_barrier()
        pltpu.sync_copy(acc.at[pl.ds(s * 3136, 3136)],
                        orefs[en].at[c, pl.ds(s * 3136, 3136)])
        plsc.subcore_barrier()


_UAT_PHASES = [("tu", "user"), ("ta", "artist"), ("tg", "tag")]


@functools.partial(
    pl.kernel,
    out_type=[jax.ShapeDtypeStruct((2, _DST_PAD[dt], 128), _F32)
              for _, dt in _UAT_PHASES],
    mesh=_MESH,
    compiler_params=pltpu.CompilerParams(use_tc_tiling_on_sc=False),
    scratch_types=[
        pltpu.VMEM_SHARED((10112, 128), _F32),
        pltpu.VMEM((64, 128), _F32),
        pltpu.VMEM((64, 128), _F32),
        pltpu.VMEM((64, 128), _F32),
        pltpu.VMEM((64, 128), _F32),
        pltpu.VMEM((80, 64), jnp.int32),
        pltpu.VMEM((80, 64), jnp.int32),
        pltpu.SemaphoreType.DMA,
        pltpu.SemaphoreType.DMA,
        pltpu.SemaphoreType.DMA,
        pltpu.SemaphoreType.DMA,
        pltpu.SemaphoreType.DMA,
        pltpu.SemaphoreType.DMA,
        pltpu.SemaphoreType.DMA,
        pltpu.SemaphoreType.DMA,
        pltpu.SemaphoreType.DMA,
    ],
)
def _sc_uat(tbl, s_tu, d_tu, s_ta, d_ta, s_tg, d_tg,
            o_tu, o_ta, o_tg,
            acc, b0, b1, b2, b3, idxs, idxd,
            sz, sg0, sg1, sg2, sg3, ss0, ss1, ss2, ss3):
    c = lax.axis_index("c")
    s = lax.axis_index("s")
    w = s * 2 + c
    bufs = [b0, b1, b2, b3]
    sgs = [sg0, sg1, sg2, sg3]
    sss = [ss0, ss1, ss2, ss3]
    srefs = {"tu": (s_tu, d_tu, o_tu), "ta": (s_ta, d_ta, o_ta),
             "tg": (s_tg, d_tg, o_tg)}
    zchunks = [(0, 64), (64, 64), (128, 64), (192, 64), (256, 64), (320, 64),
               (384, 64), (448, 64), (512, 64), (576, 56)]
    _zero_fill(b0, 64, 128)
    _zero_acc(b0, acc, s * 632, zchunks, sz)
    plsc.subcore_barrier()
    for en, dt in _UAT_PHASES:
        sref, dref, oref = srefs[en]
        nbt = _NB[en] // 32
        pltpu.sync_copy(sref.at[w], idxs.at[pl.ds(0, nbt)])
        pltpu.sync_copy(dref.at[w], idxd.at[pl.ds(0, nbt)])
        _pipe(tbl, acc, idxs, idxd, nbt, bufs, sgs, sss)
        plsc.subcore_barrier()
        rpt = _DST_PAD[dt] // 16
        pltpu.sync_copy(acc.at[pl.ds(s * rpt, rpt)],
                        oref.at[c, pl.ds(s * rpt, rpt)])
        plsc.subcore_barrier()


_TRK_PHASES = ["ut", "at", "gt"]


@functools.partial(
    pl.kernel,
    out_type=[jax.ShapeDtypeStruct((4, 50176, 32), _F32) for _ in _TRK_PHASES],
    mesh=_MESH,
    compiler_params=pltpu.CompilerParams(use_tc_tiling_on_sc=False),
    scratch_types=[
        pltpu.VMEM_SHARED((50176, 32), _F32),
        pltpu.VMEM((64, 32), _F32),
        pltpu.VMEM((64, 32), _F32),
        pltpu.VMEM((64, 32), _F32),
        pltpu.VMEM((64, 32), _F32),
        pltpu.VMEM((160, 64), jnp.int32),
        pltpu.VMEM((160, 64), jnp.int32),
        pltpu.SemaphoreType.DMA,
        pltpu.SemaphoreType.DMA,
        pltpu.SemaphoreType.DMA,
        pltpu.SemaphoreType.DMA,
        pltpu.SemaphoreType.DMA,
        pltpu.SemaphoreType.DMA,
        pltpu.SemaphoreType.DMA,
        pltpu.SemaphoreType.DMA,
        pltpu.SemaphoreType.DMA,
    ],
)
def _sc_track(tbl_u, tbl_a, tbl_g,
              s_ut, d_ut, s_at, d_at, s_gt, d_gt,
              o_ut, o_at, o_gt,
              acc, b0, b1, b2, b3, idxs, idxd,
              sz, sg0, sg1, sg2, sg3, ss0, ss1, ss2, ss3):
    c = lax.axis_index("c")
    s = lax.axis_index("s")
    bufs = [b0, b1, b2, b3]
    sgs = [sg0, sg1, sg2, sg3]
    sss = [ss0, ss1, ss2, ss3]
    refs = {"ut": (tbl_u, s_ut, d_ut, o_ut), "at": (tbl_a, s_at, d_at, o_at),
            "gt": (tbl_g, s_gt, d_gt, o_gt)}
    zchunks = [(i * 64, 64) for i in range(49)]
    _zero_fill(b0, 64, 32)
    _zero_acc(b0, acc, s * 3136, zchunks, sz)
    plsc.subcore_barrier()
    for en in _TRK_PHASES:
        tbl, sref, dref, oref = refs[en]
        nbt = _NB[en] // 16
        pltpu.sync_copy(dref.at[s], idxd.at[pl.ds(0, nbt)])
        for j in range(2):
            q = 2 * c + j
            pltpu.sync_copy(sref.at[q, s], idxs.at[pl.ds(0, nbt)])
            _pipe(tbl, acc, idxs, idxd, nbt, bufs, sgs, sss)
            plsc.subcore_barrier()
            pltpu.sync_copy(acc.at[pl.ds(s * 3136, 3136)],
                            oref.at[q, pl.ds(s * 3136, 3136)])
            plsc.subcore_barrier()


_BLK = 1024


def _simple_body(relu, has_pred, *refs):
    if has_pred:
        s0, s1, p0, p1, cinv, x, wl, wr, b, out = refs
        m = (s0[0] + s1[0] - p0[0] - p1[0]) * cinv[...]
    else:
        s0, s1, cinv, x, wl, wr, b, out = refs
        m = (s0[0] + s1[0]) * cinv[...]
    acc = (jnp.dot(m, wl[...], preferred_element_type=_F32)
           + jnp.dot(x[...], wr[...], preferred_element_type=_F32) + b[...])
    if relu:
        acc = jnp.maximum(acc, 0.0)
    out[...] = acc


def _combine_simple(s, pred, cinv, x, wl, wr, b, relu):
    n, d = x.shape
    grid = (pl.cdiv(n, _BLK),)
    row = pl.BlockSpec((_BLK, d), lambda i: (i, 0))
    sspec = [pl.BlockSpec((1, _BLK, d), lambda i: (0, i, 0)),
             pl.BlockSpec((1, _BLK, d), lambda i: (1, i, 0))]
    args = [s, s] + ([pred, pred] if pred is not None else [])
    return pl.pallas_call(
        functools.partial(_simple_body, relu, pred is not None),
        grid=grid,
        in_specs=(sspec * (2 if pred is not None else 1)
                  + [pl.BlockSpec((_BLK, 1), lambda i: (i, 0)),
                     row,
                     pl.BlockSpec((d, d), lambda i: (0, 0)),
                     pl.BlockSpec((d, d), lambda i: (0, 0)),
                     pl.BlockSpec((1, d), lambda i: (0, 0))]),
        out_specs=row,
        out_shape=jax.ShapeDtypeStruct((n, d), _F32),
    )(*args, cinv, x, wl, wr, b)


_TRK_PRED = {
    "ut": [None, ("ut", 0), None, ("ut", 2)],
    "at": [("ut", 1), ("at", 0), ("ut", 3), ("at", 2)],
    "gt": [("at", 1), ("gt", 0), ("at", 3), ("gt", 2)],
}


def _track_body(relu, *refs):
    pos = refs[0:12]
    pred = refs[12:24]
    c_ut, c_at, c_gt, x, wl_ut, wl_at, wl_gt, wr, b, out = refs[24:]
    acc = (jnp.dot(x[...], wr[...], preferred_element_type=_F32) + b[...])
    for e, (en, cinv, wl) in enumerate((("ut", c_ut, wl_ut),
                                        ("at", c_at, wl_at),
                                        ("gt", c_gt, wl_gt))):
        qs = []
        for qi in range(4):
            v = pos[e * 4 + qi][0]
            if _TRK_PRED[en][qi] is not None:
                v = v - pred[e * 4 + qi][0]
            qs.append(v)
        m = jnp.concatenate(qs, axis=1) * cinv[...]
        acc = acc + jnp.dot(m, wl[...], preferred_element_type=_F32)
    acc = acc * (1.0 / 3.0)
    if relu:
        acc = jnp.maximum(acc, 0.0)
    out[...] = acc


def _combine_track(s_ut, s_at, s_gt, c_ut, c_at, c_gt, x,
                   wl_ut, wl_at, wl_gt, wr, b, relu):
    n, d = x.shape
    grid = (pl.cdiv(n, _BLK),)
    row = pl.BlockSpec((_BLK, d), lambda i: (i, 0))
    arrs = {"ut": s_ut, "at": s_at, "gt": s_gt}
    def qspec(q):
        return pl.BlockSpec((1, _BLK, 32), lambda i, q=q: (q, i, 0))
    pos_specs, pos_args, pred_specs, pred_args = [], [], [], []
    for en in ("ut", "at", "gt"):
        for qi in range(4):
            pos_specs.append(qspec(qi))
            pos_args.append(arrs[en])
            p = _TRK_PRED[en][qi]
            if p is None:
                pred_specs.append(qspec(0))
                pred_args.append(arrs[en])
            else:
                pred_specs.append(qspec(p[1]))
                pred_args.append(arrs[p[0]])
    cspec = pl.BlockSpec((_BLK, 1), lambda i: (i, 0))
    wspec = pl.BlockSpec((d, d), lambda i: (0, 0))
    return pl.pallas_call(
        functools.partial(_track_body, relu),
        grid=grid,
        in_specs=(pos_specs + pred_specs
                  + [cspec, cspec, cspec, row, wspec, wspec, wspec, wspec,
                     pl.BlockSpec((1, d), lambda i: (0, 0))]),
        out_specs=row,
        out_shape=jax.ShapeDtypeStruct((n, d), _F32),
    )(*pos_args, *pred_args,
      c_ut, c_at, c_gt, x, wl_ut, wl_at, wl_gt, wr, b)


def _pad2d(a, epad, lo, hi):
    n = epad - a.shape[0]
    pad = lo + jnp.arange(n, dtype=jnp.int32) % (hi - lo)
    return jnp.concatenate([a, pad]).reshape(-1, 64)


def _colsplit(x):
    v = x.shape[0]
    return x.reshape(v, 4, 32).transpose(1, 0, 2).reshape(4 * v, 32)


def kernel(params, user_node_id, track_node_id, artist_node_id, tag_node_id,
           ei_ut, ei_tu, ei_ta, ei_at, ei_tg, ei_gt):
    x = {nt: params["emb_" + nt] for nt in ("user", "track", "artist", "tag")}
    ei = {"ut": ei_ut, "tu": ei_tu, "ta": ei_ta, "at": ei_at,
          "tg": ei_tg, "gt": ei_gt}

    s32, d32, d16, s16q = {}, {}, {}, {}
    for en, srct, dstt in _EDGES:
        sp = _pad2d(ei[en][0], _EPAD[en], 0, _NUM[srct])
        dp = _pad2d(ei[en][1], _EPAD[en], _NUM[dstt], _DST_PAD[dstt])
        s32[en] = sp.reshape(32, -1, 64)
        d32[en] = dp.reshape(32, -1, 64)
        if dstt == "track":
            d16[en] = dp.reshape(16, -1, 64)
            v = _NUM[srct]
            s16q[en] = (sp.reshape(16, -1, 64)[None]
                        + (jnp.arange(4, dtype=jnp.int32) * v)[:, None, None,
                                                               None])

    z16 = jnp.zeros((196, 16), _F32)
    o16 = jnp.ones((64, 16), _F32)

    cnts = _sc_counts(z16, o16, d32["ut"], d32["tu"], d32["ta"], d32["at"],
                      d32["tg"], d32["gt"])
    cinv = {}
    prev = None
    for (en, _, dstt), carr in zip(_EDGES, cnts):
        cur = carr[:, :, 0]
        dcnt = cur if prev is None else cur - prev
        prev = cur
        cnt = (dcnt[0] + dcnt[1])[:_NUM[dstt]]
        cinv[en] = (1.0 / jnp.maximum(cnt, 1.0)).reshape(-1, 1)

    for l in range(2):
        relu = l == 0
        s_ut, s_at, s_gt = _sc_track(
            _colsplit(x["user"]), _colsplit(x["artist"]),
            _colsplit(x["tag"]),
            s16q["ut"], d16["ut"], s16q["at"], d16["at"], s16q["gt"],
            d16["gt"])
        s_tu, s_ta, s_tg = _sc_uat(
            x["track"], s32["tu"], d32["tu"], s32["ta"], d32["ta"],
            s32["tg"], d32["tg"])
        new_x = {}
        new_x["user"] = _combine_simple(
            s_tu, None, cinv["tu"], x["user"], params["l%d_tu_Wl" % l],
            params["l%d_tu_Wr" % l], params["l%d_tu_bl" % l].reshape(1, -1),
            relu)
        new_x["artist"] = _combine_simple(
            s_ta, s_tu, cinv["ta"], x["artist"], params["l%d_ta_Wl" % l],
            params["l%d_ta_Wr" % l], params["l%d_ta_bl" % l].reshape(1, -1),
            relu)
        new_x["tag"] = _combine_simple(
            s_tg, s_ta, cinv["tg"], x["tag"], params["l%d_tg_Wl" % l],
            params["l%d_tg_Wr" % l], params["l%d_tg_bl" % l].reshape(1, -1),
            relu)
        wr_sum = (params["l%d_ut_Wr" % l] + params["l%d_at_Wr" % l]
                  + params["l%d_gt_Wr" % l])
        b_sum = (params["l%d_ut_bl" % l] + params["l%d_at_bl" % l]
                 + params["l%d_gt_bl" % l]).reshape(1, -1)
        new_x["track"] = _combine_track(
            s_ut, s_at, s_gt, cinv["ut"], cinv["at"], cinv["gt"], x["track"],
            params["l%d_ut_Wl" % l], params["l%d_at_Wl" % l],
            params["l%d_gt_Wl" % l], wr_sum, b_sum, relu)
        x = new_x
    return (x["user"], x["track"], x["artist"], x["tag"])
